# Initial kernel scaffold; baseline (speedup 1.0000x reference)
import jax
import jax.numpy as jnp
from jax.experimental import pallas as pl


def kernel(x, offsets, lin_table, lin_bias, ffm_tables):
    raise NotImplementedError("write your pallas kernel here")


# speedup vs baseline: 13.4726x; 13.4726x over previous
import functools

import jax
import jax.numpy as jnp
from jax import lax
from jax.experimental import pallas as pl
from jax.experimental.pallas import tpu as pltpu
from jax.experimental.pallas import tpu_sc as plsc

_NW = 32
_NC = 2
_LANES = 16

_F = 26
_D = 32
_PAIR_SLOTS = _F * _F
_SLOT_PAD = 680
_LIN_SLOT = _SLOT_PAD
_ROW_PAD = 768
_NCHUNK_FULL = 5
_TAIL_ROWS = 40
_LIN_ROWS = 32


def _ffm_body(rows_hbm, lin_hbm, ffm_hbm, out_hbm, idxv, a_v, lin_v, out_v,
              acc_v, sem_i, sem_a, *, bpw):
    wid = lax.axis_index("s") * _NC + lax.axis_index("c")
    base = wid * bpw

    def issue_gathers(buf):
        for c in range(_NCHUNK_FULL):
            pltpu.async_copy(
                ffm_hbm.at[idxv.at[buf, c]],
                a_v.at[buf, pl.ds(c * 128, 128), :],
                sem_a.at[buf],
            )
        pltpu.async_copy(
            ffm_hbm.at[idxv.at[buf, _NCHUNK_FULL, pl.ds(0, _TAIL_ROWS)]],
            a_v.at[buf, pl.ds(_NCHUNK_FULL * 128, _TAIL_ROWS), :],
            sem_a.at[buf],
        )
        pltpu.async_copy(
            lin_hbm.at[idxv.at[buf, _NCHUNK_FULL, pl.ds(_TAIL_ROWS, _LIN_ROWS)]],
            lin_v.at[buf],
            sem_a.at[buf],
        )

    def wait_gathers(buf):
        for c in range(_NCHUNK_FULL):
            pltpu.make_async_copy(
                ffm_hbm.at[idxv.at[buf, c]],
                a_v.at[buf, pl.ds(c * 128, 128), :],
                sem_a.at[buf],
            ).wait()
        pltpu.make_async_copy(
            ffm_hbm.at[idxv.at[buf, _NCHUNK_FULL, pl.ds(0, _TAIL_ROWS)]],
            a_v.at[buf, pl.ds(_NCHUNK_FULL * 128, _TAIL_ROWS), :],
            sem_a.at[buf],
        ).wait()
        pltpu.make_async_copy(
            lin_hbm.at[idxv.at[buf, _NCHUNK_FULL, pl.ds(_TAIL_ROWS, _LIN_ROWS)]],
            lin_v.at[buf],
            sem_a.at[buf],
        ).wait()

    lane = lax.broadcasted_iota(jnp.int32, (_LANES,), 0)

    def compute(buf, s):
        l0 = lin_v[buf, pl.ds(0, _LANES)]
        l1 = lin_v[buf, pl.ds(_LANES, _LANES)]
        acc0 = l0
        acc1 = jnp.where(lane < _F - _LANES, l1, 0.0)
        for i in range(_F):
            for j in range(i + 1, _F):
                ui = i * _F + j
                vi = j * _F + i
                u0 = a_v[buf, ui, pl.ds(0, _LANES)]
                v0 = a_v[buf, vi, pl.ds(0, _LANES)]
                acc0 = acc0 + u0 * v0
                u1 = a_v[buf, ui, pl.ds(_LANES, _LANES)]
                v1 = a_v[buf, vi, pl.ds(_LANES, _LANES)]
                acc1 = acc1 + u1 * v1
        acc_v[pl.ds(lax.rem(s, _LANES) * _LANES, _LANES)] = acc0 + acc1

    def step(t, buf):
        s = 2 * t + buf
        nbuf = 1 - buf
        wait_gathers(buf)

        @pl.when(s + 2 < bpw)
        def _():
            pltpu.async_copy(rows_hbm.at[base + s + 2], idxv.at[buf],
                             sem_i.at[buf])

        @pl.when(s + 1 < bpw)
        def _():
            pltpu.make_async_copy(rows_hbm.at[base], idxv.at[nbuf],
                                  sem_i.at[nbuf]).wait()
            issue_gathers(nbuf)

        compute(buf, s)

    pltpu.sync_copy(rows_hbm.at[base], idxv.at[0])
    issue_gathers(0)
    pltpu.async_copy(rows_hbm.at[base + 1], idxv.at[1], sem_i.at[1])

    def body(t, carry):
        step(t, 0)
        step(t, 1)

        @pl.when(lax.rem(t, 8) == 7)
        def _():
            total = jnp.zeros((_LANES,), jnp.float32)
            for l in range(_LANES):
                total = total + plsc.load_gather(acc_v, [lane * _LANES + l])
            sig = 1.0 / (1.0 + jnp.exp(-total))
            out_v[pl.ds(2 * t - 14, _LANES)] = sig

        return carry

    lax.fori_loop(0, bpw // 2, body, jnp.int32(0))
    pltpu.sync_copy(out_v, out_hbm.at[pl.ds(base, bpw)])


def kernel(x, offsets, lin_table, lin_bias, ffm_tables):
    b, f = x.shape
    fv, v, d = ffm_tables.shape
    assert f == _F and d == _D and b % _NW == 0
    bpw = b // _NW

    idx = x + offsets[None, :]
    tbl = jnp.arange(_F, dtype=jnp.int32) * v
    ffm_ids = idx[:, :, None] + tbl[None, None, :]
    rows = jnp.concatenate(
        [
            ffm_ids.reshape(b, _PAIR_SLOTS),
            jnp.zeros((b, _SLOT_PAD - _PAIR_SLOTS), jnp.int32),
            idx,
            jnp.zeros((b, _ROW_PAD - _LIN_SLOT - _F), jnp.int32),
        ],
        axis=1,
    ).reshape(b, _ROW_PAD // 128, 128)

    lin_flat = lin_table[:, 0] + lin_bias[0] / _F
    ffm_flat = ffm_tables.reshape(fv * v, d)

    mesh = plsc.VectorSubcoreMesh(core_axis_name="c", subcore_axis_name="s")
    run = pl.kernel(
        functools.partial(_ffm_body, bpw=bpw),
        out_type=jax.ShapeDtypeStruct((b,), jnp.float32),
        mesh=mesh,
        compiler_params=pltpu.CompilerParams(
            needs_layout_passes=False, use_tc_tiling_on_sc=False),
        scratch_types=[
            pltpu.VMEM((2, _ROW_PAD // 128, 128), jnp.int32),
            pltpu.VMEM((2, _ROW_PAD, _D), jnp.float32),
            pltpu.VMEM((2, _LIN_ROWS), jnp.float32),
            pltpu.VMEM((bpw,), jnp.float32),
            pltpu.VMEM((_LANES * _LANES,), jnp.float32),
            pltpu.SemaphoreType.DMA((2,)),
            pltpu.SemaphoreType.DMA((2,)),
        ],
    )
    return run(rows, lin_flat, ffm_flat)
